# Initial kernel scaffold; baseline (speedup 1.0000x reference)
import jax
import jax.numpy as jnp
from jax.experimental import pallas as pl


def kernel(x, label, units):
    raise NotImplementedError("write your pallas kernel here")


# speedup vs baseline: 1.7845x; 1.7845x over previous
import jax
import jax.numpy as jnp
from jax.experimental import pallas as pl
from jax.experimental.pallas import tpu as pltpu

MAR = 0.999


def _colnorm(v):
    n = jnp.sqrt(jnp.sum(v * v, axis=0, keepdims=True))
    return v / jnp.maximum(n, 1e-12)


def _rownorm(v):
    n = jnp.sqrt(jnp.sum(v * v, axis=1, keepdims=True))
    return v / jnp.maximum(n, 1e-12)


def _update_attn_kernel(label_ref, prev_ref, xt_ref, units_ref,
                        mem_ref, ot_ref, mem_scr):
    i = pl.program_id(0)
    xt = xt_ref[0]
    mm0 = units_ref[0]
    k = mm0.shape[0]
    n = xt.shape[1]
    p = prev_ref[i]
    mm_prev = mem_scr[jnp.maximum(p, 0)]
    mm = jnp.where(p >= 0, mm_prev, mm0)

    xtn = _colnorm(xt)
    mmn = _rownorm(mm)
    score = jax.lax.dot_general(xtn, mmn, (((0,), (1,)), ((), ())),
                                preferred_element_type=jnp.float32)
    mx = jnp.max(score, axis=1, keepdims=True)
    iota_k = jax.lax.broadcasted_iota(jnp.int32, (n, k), 1)
    ind = jnp.min(jnp.where(score >= mx, iota_k, k), axis=1, keepdims=True)
    onehot = (iota_k == ind).astype(jnp.float32)
    counts = jnp.sum(onehot, axis=0)
    embed = jax.lax.dot_general(onehot, xt, (((0,), (1,)), ((), ())),
                                preferred_element_type=jnp.float32)
    scale = (1.0 - MAR) / (counts + 1e-8)
    new = mm * MAR + embed * scale[:, None]
    mem_scr[i] = new
    mem_ref[0] = new

    mn = _rownorm(new)
    score2 = jax.lax.dot_general(xtn, mn, (((0,), (1,)), ((), ())),
                                 preferred_element_type=jnp.float32)
    score2 = score2 - jnp.max(score2, axis=1, keepdims=True)
    e = jnp.exp(score2)
    pattn = e / jnp.sum(e, axis=1, keepdims=True)
    ot_ref[0] = jax.lax.dot_general(new, pattn, (((0,), (1,)), ((), ())),
                                    preferred_element_type=jnp.float32)


def _assemble_kernel(sel_ref, used_ref, units_ref, mem_ref, m_ref):
    a = pl.program_id(0)
    m_ref[0] = jnp.where(used_ref[a] > 0, mem_ref[0], units_ref[0])


def kernel(x, label, units):
    b, c, s, h, w = x.shape
    a, k, cdim = units.shape
    n = h * w
    xt = x.reshape(b, cdim, n)

    idx = jnp.arange(b, dtype=jnp.int32)
    eq = label[None, :] == label[:, None]
    lower = idx[None, :] < idx[:, None]
    prev = jnp.max(jnp.where(eq & lower, idx[None, :], -1), axis=1)

    cls = jnp.arange(a, dtype=jnp.int32)
    match = label[None, :] == cls[:, None]
    sel = jnp.max(jnp.where(match, idx[None, :], -1), axis=1)
    used = (sel >= 0).astype(jnp.int32)
    sel = jnp.maximum(sel, 0)

    grid_spec = pltpu.PrefetchScalarGridSpec(
        num_scalar_prefetch=2,
        grid=(b,),
        in_specs=[
            pl.BlockSpec((1, cdim, n), lambda i, lab, prv: (i, 0, 0)),
            pl.BlockSpec((1, k, cdim), lambda i, lab, prv: (lab[i], 0, 0)),
        ],
        out_specs=[
            pl.BlockSpec((1, k, cdim), lambda i, lab, prv: (i, 0, 0)),
            pl.BlockSpec((1, cdim, n), lambda i, lab, prv: (i, 0, 0)),
        ],
        scratch_shapes=[pltpu.VMEM((b, k, cdim), jnp.float32)],
    )
    mem, ot = pl.pallas_call(
        _update_attn_kernel,
        grid_spec=grid_spec,
        out_shape=[
            jax.ShapeDtypeStruct((b, k, cdim), x.dtype),
            jax.ShapeDtypeStruct((b, cdim, n), x.dtype),
        ],
    )(label, prev, xt, units)

    grid_spec2 = pltpu.PrefetchScalarGridSpec(
        num_scalar_prefetch=2,
        grid=(a,),
        in_specs=[
            pl.BlockSpec((1, k, cdim), lambda j, se, us: (j, 0, 0)),
            pl.BlockSpec((1, k, cdim), lambda j, se, us: (se[j], 0, 0)),
        ],
        out_specs=pl.BlockSpec((1, k, cdim), lambda j, se, us: (j, 0, 0)),
    )
    m = pl.pallas_call(
        _assemble_kernel,
        grid_spec=grid_spec2,
        out_shape=jax.ShapeDtypeStruct((a, k, cdim), units.dtype),
    )(sel, used, units, mem)

    out = ot.reshape(b, c, s, h, w)
    return (m, out)


# speedup vs baseline: 1.8007x; 1.0091x over previous
import jax
import jax.numpy as jnp
from jax.experimental import pallas as pl
from jax.experimental.pallas import tpu as pltpu

MAR = 0.999


def _colnorm(v):
    n = jnp.sqrt(jnp.sum(v * v, axis=0, keepdims=True))
    return v / jnp.maximum(n, 1e-12)


def _rownorm(v):
    n = jnp.sqrt(jnp.sum(v * v, axis=1, keepdims=True))
    return v / jnp.maximum(n, 1e-12)


def _fused_kernel(label_ref, prev_ref, sel_ref, used_ref, ord_ref,
                  xt_ref, units_ref, unitsb_ref,
                  mem_ref, ot_ref, m_ref, mem_scr):
    t = pl.program_id(0)
    nb = mem_scr.shape[0]

    @pl.when(t < nb)
    def _update_and_attend():
        xt = xt_ref[0]
        mm0 = units_ref[0]
        k = mm0.shape[0]
        n = xt.shape[1]
        p = prev_ref[t]
        mm_prev = mem_scr[jnp.maximum(p, 0)]
        mm = jnp.where(p >= 0, mm_prev, mm0)

        xtn = _colnorm(xt)
        mmn = _rownorm(mm)
        score = jax.lax.dot_general(
            xtn.astype(jnp.bfloat16), mmn.astype(jnp.bfloat16),
            (((0,), (1,)), ((), ())), preferred_element_type=jnp.float32)
        mx = jnp.max(score, axis=1, keepdims=True)
        iota_k = jax.lax.broadcasted_iota(jnp.int32, (n, k), 1)
        ind = jnp.min(jnp.where(score >= mx, iota_k, k), axis=1, keepdims=True)
        onehot = (iota_k == ind).astype(jnp.bfloat16)
        counts = jnp.sum(onehot.astype(jnp.float32), axis=0)
        embed = jax.lax.dot_general(
            onehot, xt.astype(jnp.bfloat16),
            (((0,), (1,)), ((), ())), preferred_element_type=jnp.float32)
        scale = (1.0 - MAR) / (counts + 1e-8)
        new = mm * MAR + embed * scale[:, None]
        mem_scr[t] = new
        mem_ref[0] = new

        mn = _rownorm(new)
        score2 = jax.lax.dot_general(xtn, mn, (((0,), (1,)), ((), ())),
                                     preferred_element_type=jnp.float32)
        score2 = score2 - jnp.max(score2, axis=1, keepdims=True)
        e = jnp.exp(score2)
        pattn = e / jnp.sum(e, axis=1, keepdims=True)
        ot_ref[0] = jax.lax.dot_general(new, pattn, (((0,), (1,)), ((), ())),
                                        preferred_element_type=jnp.float32)

    a = ord_ref[t]
    fin = mem_scr[sel_ref[a]]
    m_ref[0] = jnp.where(used_ref[a] > 0, fin, unitsb_ref[0])


def kernel(x, label, units):
    b, c, s, h, w = x.shape
    a, k, cdim = units.shape
    n = h * w
    xt = x.reshape(b, cdim, n)

    idx = jnp.arange(b, dtype=jnp.int32)
    eq = label[None, :] == label[:, None]
    lower = idx[None, :] < idx[:, None]
    prev = jnp.max(jnp.where(eq & lower, idx[None, :], -1), axis=1)

    cls = jnp.arange(a, dtype=jnp.int32)
    match = label[None, :] == cls[:, None]
    sel = jnp.max(jnp.where(match, idx[None, :], -1), axis=1)
    used = (sel >= 0).astype(jnp.int32)
    sel = jnp.maximum(sel, 0)
    order = jnp.argsort(used, stable=True).astype(jnp.int32)

    bm1 = b - 1
    grid_spec = pltpu.PrefetchScalarGridSpec(
        num_scalar_prefetch=5,
        grid=(a,),
        in_specs=[
            pl.BlockSpec((1, cdim, n),
                         lambda t, lab, prv, se, us, od: (jnp.minimum(t, bm1), 0, 0)),
            pl.BlockSpec((1, k, cdim),
                         lambda t, lab, prv, se, us, od: (lab[jnp.minimum(t, bm1)], 0, 0)),
            pl.BlockSpec((1, k, cdim),
                         lambda t, lab, prv, se, us, od: (od[t], 0, 0)),
        ],
        out_specs=[
            pl.BlockSpec((1, k, cdim),
                         lambda t, lab, prv, se, us, od: (jnp.minimum(t, bm1), 0, 0)),
            pl.BlockSpec((1, cdim, n),
                         lambda t, lab, prv, se, us, od: (jnp.minimum(t, bm1), 0, 0)),
            pl.BlockSpec((1, k, cdim),
                         lambda t, lab, prv, se, us, od: (od[t], 0, 0)),
        ],
        scratch_shapes=[pltpu.VMEM((b, k, cdim), jnp.float32)],
    )
    mem, ot, m = pl.pallas_call(
        _fused_kernel,
        grid_spec=grid_spec,
        out_shape=[
            jax.ShapeDtypeStruct((b, k, cdim), x.dtype),
            jax.ShapeDtypeStruct((b, cdim, n), x.dtype),
            jax.ShapeDtypeStruct((a, k, cdim), units.dtype),
        ],
    )(label, prev, sel, used, order, xt, units, units)

    out = ot.reshape(b, c, s, h, w)
    return (m, out)


# speedup vs baseline: 1.8115x; 1.0060x over previous
import jax
import jax.numpy as jnp
from jax.experimental import pallas as pl
from jax.experimental.pallas import tpu as pltpu

MAR = 0.999


def _rownorm(v):
    n = jnp.sqrt(jnp.sum(v * v, axis=1, keepdims=True))
    return v / jnp.maximum(n, 1e-12)


def _fused_kernel(label_ref, prev_ref, sel_ref, used_ref, ord_ref,
                  xv_ref, units_ref, unitsb_ref, p_ref,
                  mem_ref, ot_ref, m_ref, mem_scr):
    t = pl.program_id(0)
    nb = mem_scr.shape[0]

    @pl.when(t < nb)
    def _update_and_attend():
        xv = xv_ref[0]
        mm0 = units_ref[0]
        P = p_ref[...]
        k = mm0.shape[0]
        s, n, c = xv.shape
        p = prev_ref[t]
        mm_prev = mem_scr[jnp.maximum(p, 0)]
        mm = jnp.where(p >= 0, mm_prev, mm0)

        ss = jnp.sum(jnp.sum(xv * xv, axis=2), axis=0)
        rn = 1.0 / jnp.maximum(jnp.sqrt(ss), 1e-12)
        xvn = xv * rn[None, :, None]
        mmn = _rownorm(mm)

        mmn_b = mmn.astype(jnp.bfloat16)
        P_b = P.astype(jnp.bfloat16)
        xvn_b = xvn.astype(jnp.bfloat16)
        score = jnp.zeros((n, k), jnp.float32)
        for si in range(s):
            psl = P_b[:, si * c:(si + 1) * c]
            mf = jax.lax.dot_general(mmn_b, psl, (((1,), (0,)), ((), ())),
                                     preferred_element_type=jnp.float32)
            score += jax.lax.dot_general(xvn_b[si], mf.astype(jnp.bfloat16),
                                         (((1,), (1,)), ((), ())),
                                         preferred_element_type=jnp.float32)
        mx = jnp.max(score, axis=1, keepdims=True)
        iota_k = jax.lax.broadcasted_iota(jnp.int32, (n, k), 1)
        ind = jnp.min(jnp.where(score >= mx, iota_k, k), axis=1, keepdims=True)
        onehot = (iota_k == ind).astype(jnp.bfloat16)
        counts = jnp.sum(onehot.astype(jnp.float32), axis=0)

        xv_b = xv.astype(jnp.bfloat16)
        embed = jnp.zeros((k, mm0.shape[1]), jnp.float32)
        for si in range(s):
            emb_s = jax.lax.dot_general(onehot, xv_b[si], (((0,), (0,)), ((), ())),
                                        preferred_element_type=jnp.float32)
            psl = P[:, si * c:(si + 1) * c]
            embed += jax.lax.dot_general(emb_s.astype(jnp.bfloat16),
                                         psl.astype(jnp.bfloat16),
                                         (((1,), (1,)), ((), ())),
                                         preferred_element_type=jnp.float32)
        scale = (1.0 - MAR) / (counts + 1e-8)
        new = mm * MAR + embed * scale[:, None]
        mem_scr[t] = new
        mem_ref[0] = new

        mn = _rownorm(new)
        score2 = jnp.zeros((n, k), jnp.float32)
        for si in range(s):
            psl = P[:, si * c:(si + 1) * c]
            mnf = jax.lax.dot_general(mn, psl, (((1,), (0,)), ((), ())),
                                      preferred_element_type=jnp.float32)
            score2 += jax.lax.dot_general(xvn[si], mnf, (((1,), (1,)), ((), ())),
                                          preferred_element_type=jnp.float32)
        score2 = score2 - jnp.max(score2, axis=1, keepdims=True)
        e = jnp.exp(score2)
        pattn = e / jnp.sum(e, axis=1, keepdims=True)
        for si in range(s):
            psl = P[:, si * c:(si + 1) * c]
            nf = jax.lax.dot_general(new, psl, (((1,), (0,)), ((), ())),
                                     preferred_element_type=jnp.float32)
            ot_ref[0, si] = jax.lax.dot_general(pattn, nf, (((1,), (0,)), ((), ())),
                                                preferred_element_type=jnp.float32)

    a = ord_ref[t]
    fin = mem_scr[sel_ref[a]]
    m_ref[0] = jnp.where(used_ref[a] > 0, fin, unitsb_ref[0])


def kernel(x, label, units):
    b, cch, s, h, w = x.shape
    a, k, cdim = units.shape
    n = h * w
    xv = jnp.transpose(x, (0, 2, 3, 4, 1)).reshape(b, s, n, cch)

    cd = jnp.arange(cdim, dtype=jnp.int32)
    cdp = (cd % s) * cch + (cd // s)
    P = (cdp[:, None] == cd[None, :]).astype(jnp.float32)

    idx = jnp.arange(b, dtype=jnp.int32)
    eq = label[None, :] == label[:, None]
    lower = idx[None, :] < idx[:, None]
    prev = jnp.max(jnp.where(eq & lower, idx[None, :], -1), axis=1)

    cls = jnp.arange(a, dtype=jnp.int32)
    match = label[None, :] == cls[:, None]
    sel = jnp.max(jnp.where(match, idx[None, :], -1), axis=1)
    used = (sel >= 0).astype(jnp.int32)
    sel = jnp.maximum(sel, 0)
    order = jnp.argsort(used, stable=True).astype(jnp.int32)

    bm1 = b - 1
    grid_spec = pltpu.PrefetchScalarGridSpec(
        num_scalar_prefetch=5,
        grid=(a,),
        in_specs=[
            pl.BlockSpec((1, s, n, cch),
                         lambda t, lab, prv, se, us, od: (jnp.minimum(t, bm1), 0, 0, 0)),
            pl.BlockSpec((1, k, cdim),
                         lambda t, lab, prv, se, us, od: (lab[jnp.minimum(t, bm1)], 0, 0)),
            pl.BlockSpec((1, k, cdim),
                         lambda t, lab, prv, se, us, od: (od[t], 0, 0)),
            pl.BlockSpec((cdim, cdim),
                         lambda t, lab, prv, se, us, od: (0, 0)),
        ],
        out_specs=[
            pl.BlockSpec((1, k, cdim),
                         lambda t, lab, prv, se, us, od: (jnp.minimum(t, bm1), 0, 0)),
            pl.BlockSpec((1, s, n, cch),
                         lambda t, lab, prv, se, us, od: (jnp.minimum(t, bm1), 0, 0, 0)),
            pl.BlockSpec((1, k, cdim),
                         lambda t, lab, prv, se, us, od: (od[t], 0, 0)),
        ],
        scratch_shapes=[pltpu.VMEM((b, k, cdim), jnp.float32)],
    )
    mem, ot, m = pl.pallas_call(
        _fused_kernel,
        grid_spec=grid_spec,
        out_shape=[
            jax.ShapeDtypeStruct((b, k, cdim), x.dtype),
            jax.ShapeDtypeStruct((b, s, n, cch), x.dtype),
            jax.ShapeDtypeStruct((a, k, cdim), units.dtype),
        ],
    )(label, prev, sel, used, order, xv, units, units, P)

    out = jnp.transpose(ot.reshape(b, s, h, w, cch), (0, 4, 1, 2, 3))
    return (m, out)


# speedup vs baseline: 3.2259x; 1.7808x over previous
import jax
import jax.numpy as jnp
from jax.experimental import pallas as pl
from jax.experimental.pallas import tpu as pltpu

MAR = 0.999
BF = jnp.bfloat16


def _rownorm(v):
    n = jnp.sqrt(jnp.sum(v * v, axis=1, keepdims=True))
    return v / jnp.maximum(n, 1e-12)


def _fused_kernel(label_ref, prev_ref, sel_ref, used_ref, ord_ref,
                  xv_ref, units_ref, unitsb_ref, p_ref,
                  ot_ref, m_ref, mem_scr):
    t = pl.program_id(0)
    nb = mem_scr.shape[0]

    @pl.when(t < nb)
    def _update_and_attend():
        xv = xv_ref[0]
        mm0 = units_ref[0]
        P_b = p_ref[...].astype(BF)
        k = mm0.shape[0]
        s, n, c = xv.shape
        p = prev_ref[t]
        mm_prev = mem_scr[jnp.maximum(p, 0)]
        mm = jnp.where(p >= 0, mm_prev, mm0)

        xx_f = jnp.concatenate([xv[si] for si in range(s)], axis=1)
        rn = 1.0 / jnp.maximum(jnp.sqrt(jnp.sum(xx_f * xx_f, axis=1,
                                                keepdims=True)), 1e-12)
        xvn_b = (xx_f * rn).astype(BF)
        mmn_b = _rownorm(mm).astype(BF)
        mmnf_b = jax.lax.dot_general(mmn_b, P_b, (((1,), (0,)), ((), ())),
                                     preferred_element_type=jnp.float32
                                     ).astype(BF)
        score = jax.lax.dot_general(xvn_b, mmnf_b, (((1,), (1,)), ((), ())),
                                    preferred_element_type=jnp.float32)
        iota_k = jax.lax.broadcasted_iota(jnp.int32, (n, k), 1)
        ki = jax.lax.bitcast_convert_type(score + 2.0, jnp.int32)
        key = jnp.bitwise_or(jnp.bitwise_and(ki, -(k)), (k - 1) - iota_k)
        kmax = jnp.max(key, axis=1, keepdims=True)
        onehot = (key == kmax).astype(BF)
        counts = jnp.sum(onehot.astype(jnp.float32), axis=0)

        embed_f = jax.lax.dot_general(onehot, xx_f.astype(BF),
                                      (((0,), (0,)), ((), ())),
                                      preferred_element_type=jnp.float32)
        embed = jax.lax.dot_general(embed_f.astype(BF), P_b,
                                    (((1,), (1,)), ((), ())),
                                    preferred_element_type=jnp.float32)
        scale = (1.0 - MAR) / (counts + 1e-8)
        new = mm * MAR + embed * scale[:, None]
        mem_scr[t] = new

        mn_b = _rownorm(new).astype(BF)
        mnf_b = jax.lax.dot_general(mn_b, P_b, (((1,), (0,)), ((), ())),
                                    preferred_element_type=jnp.float32
                                    ).astype(BF)
        score2 = jax.lax.dot_general(xvn_b, mnf_b, (((1,), (1,)), ((), ())),
                                     preferred_element_type=jnp.float32)
        e = jnp.exp(score2)
        rs = 1.0 / jnp.sum(e, axis=1, keepdims=True)
        newf_b = jax.lax.dot_general(new.astype(BF), P_b,
                                     (((1,), (0,)), ((), ())),
                                     preferred_element_type=jnp.float32
                                     ).astype(BF)
        out_f = jax.lax.dot_general(e.astype(BF), newf_b,
                                    (((1,), (0,)), ((), ())),
                                    preferred_element_type=jnp.float32) * rs
        for si in range(s):
            ot_ref[0, si] = out_f[:, si * c:(si + 1) * c]

    a = ord_ref[t]
    fin = mem_scr[sel_ref[a]]
    m_ref[0] = jnp.where(used_ref[a] > 0, fin, unitsb_ref[0])


def kernel(x, label, units):
    b, cch, s, h, w = x.shape
    a, k, cdim = units.shape
    n = h * w
    xv = jnp.transpose(x, (0, 2, 3, 4, 1)).reshape(b, s, n, cch)

    cd = jnp.arange(cdim, dtype=jnp.int32)
    cdp = (cd % s) * cch + (cd // s)
    P = (cdp[:, None] == cd[None, :]).astype(jnp.float32)

    idx = jnp.arange(b, dtype=jnp.int32)
    eq = label[None, :] == label[:, None]
    lower = idx[None, :] < idx[:, None]
    prev = jnp.max(jnp.where(eq & lower, idx[None, :], -1), axis=1)

    cls = jnp.arange(a, dtype=jnp.int32)
    match = label[None, :] == cls[:, None]
    sel = jnp.max(jnp.where(match, idx[None, :], -1), axis=1)
    used = (sel >= 0).astype(jnp.int32)
    sel = jnp.maximum(sel, 0)
    order = jnp.argsort(used, stable=True).astype(jnp.int32)

    bm1 = b - 1
    grid_spec = pltpu.PrefetchScalarGridSpec(
        num_scalar_prefetch=5,
        grid=(a,),
        in_specs=[
            pl.BlockSpec((1, s, n, cch),
                         lambda t, lab, prv, se, us, od: (jnp.minimum(t, bm1), 0, 0, 0)),
            pl.BlockSpec((1, k, cdim),
                         lambda t, lab, prv, se, us, od: (lab[jnp.minimum(t, bm1)], 0, 0)),
            pl.BlockSpec((1, k, cdim),
                         lambda t, lab, prv, se, us, od: (od[t], 0, 0)),
            pl.BlockSpec((cdim, cdim),
                         lambda t, lab, prv, se, us, od: (0, 0)),
        ],
        out_specs=[
            pl.BlockSpec((1, s, n, cch),
                         lambda t, lab, prv, se, us, od: (jnp.minimum(t, bm1), 0, 0, 0)),
            pl.BlockSpec((1, k, cdim),
                         lambda t, lab, prv, se, us, od: (od[t], 0, 0)),
        ],
        scratch_shapes=[pltpu.VMEM((b, k, cdim), jnp.float32)],
    )
    ot, m = pl.pallas_call(
        _fused_kernel,
        grid_spec=grid_spec,
        out_shape=[
            jax.ShapeDtypeStruct((b, s, n, cch), x.dtype),
            jax.ShapeDtypeStruct((a, k, cdim), units.dtype),
        ],
    )(label, prev, sel, used, order, xv, units, units, P)

    out = jnp.transpose(ot.reshape(b, s, h, w, cch), (0, 4, 1, 2, 3))
    return (m, out)
